# baseline (speedup 1.0000x reference)
import dataclasses
import functools

import jax
import jax.numpy as jnp
from jax import lax
from jax.experimental import pallas as pl
from jax.experimental.pallas import tpu as pltpu
from jax.experimental.pallas import tpu_sc as plsc

_N = 10000
_T = 20000
_E = 320000
_C = 128
_OUT = 10

_NC = 2
_NS = 16
_NW = _NC * _NS

_NP = 10240
_ROWS_SUB = _NP // _NS
_XPW = _NP // _NW
_XCH = 80
_ECH = 128
_EPW = 10240
_EP = _EPW * _NW
_NCH = _EPW // _ECH
_DW = 128
_ZR = 64

_BLK = 1024
_GRID = _NP // _BLK


@functools.cache
def _get_mesh():
    return plsc.VectorSubcoreMesh(core_axis_name="c", subcore_axis_name="s",
                                  num_cores=_NC, num_subcores=_NS)


@functools.cache
def _get_gather_x_deg():
    @functools.partial(
        pl.kernel,
        out_type=[
            jax.ShapeDtypeStruct((_NP, _C), jnp.float32),
            jax.ShapeDtypeStruct((_NC * _NP,), jnp.float32),
        ],
        mesh=_get_mesh(),
        scratch_types=[
            pltpu.VMEM((_XCH,), jnp.int32),
            pltpu.VMEM((_XCH, _C), jnp.float32),
            pltpu.VMEM((_ECH,), jnp.int32),
            pltpu.VMEM((_ECH,), jnp.int32),
            pltpu.VMEM((_NP,), jnp.float32),
            pltpu.VMEM((_ROWS_SUB,), jnp.float32),
            pltpu.VMEM((_ROWS_SUB,), jnp.float32),
            pltpu.VMEM_SHARED((_NS * _NP,), jnp.float32),
            pltpu.SemaphoreType.DMA,
            pltpu.SemaphoreType.DMA,
            pltpu.SemaphoreType.DMA,
        ],
        compiler_params=dataclasses.replace(pltpu.CompilerParams(),
                                            needs_layout_passes=False),
    )
    def _sc_gather_x_deg(tbl_hbm, nidx_hbm, dst_hbm, x_hbm, deg_hbm,
                         idx_v, rows_v, d0, d1, hbuf, rbuf, abuf, hstage,
                         sem, isem0, isem1):
        cid = lax.axis_index("c")
        sid = lax.axis_index("s")
        wid = sid * _NC + cid

        z = jnp.zeros((16,), jnp.float32)
        o = jnp.ones((16,), jnp.float32)

        @pl.loop(0, _NP // 16)
        def _(i):
            hbuf[pl.ds(i * 16, 16)] = z

        base = wid * _XPW
        for j in range(_XPW // _XCH):
            off = base + j * _XCH
            pltpu.sync_copy(nidx_hbm.at[pl.ds(off, _XCH)], idx_v)
            pltpu.async_copy(tbl_hbm.at[idx_v], rows_v, sem).wait()
            pltpu.sync_copy(rows_v, x_hbm.at[pl.ds(off, _XCH)])

        ebase = wid * _EPW
        elast = ebase + _EPW - _ECH

        def load_dst(off, buf, isem):
            pltpu.async_copy(
                dst_hbm.at[pl.ds(lax.min(off, elast), _ECH)], buf, isem)

        load_dst(ebase, d0, isem0)
        load_dst(ebase + _ECH, d1, isem1)

        def half(off, da, isa, db, isb):
            pltpu.make_async_copy(dst_hbm.at[pl.ds(ebase, _ECH)], da,
                                  isa).wait()
            for j in range(_ECH // 16):
                plsc.addupdate_scatter(hbuf, [da[pl.ds(j * 16, 16)]], o)
            load_dst(off + 2 * _ECH, da, isa)

        @pl.loop(0, _NCH // 2)
        def _(i):
            off = ebase + 2 * i * _ECH
            half(off, d0, isem0, d1, isem1)
            half(off + _ECH, d1, isem1, d0, isem0)

        pltpu.make_async_copy(dst_hbm.at[pl.ds(ebase, _ECH)], d0, isem0).wait()
        pltpu.make_async_copy(dst_hbm.at[pl.ds(ebase, _ECH)], d1, isem1).wait()

        pltpu.sync_copy(hbuf, hstage.at[pl.ds(sid * _NP, _NP)])
        plsc.subcore_barrier()

        rbase = sid * _ROWS_SUB

        @pl.loop(0, _ROWS_SUB // 16)
        def _(i):
            abuf[pl.ds(i * 16, 16)] = z

        @pl.loop(0, _NS)
        def _(k):
            pltpu.sync_copy(hstage.at[pl.ds(k * _NP + rbase, _ROWS_SUB)],
                            rbuf)
            for t in range(_ROWS_SUB // 16):
                sl = pl.ds(t * 16, 16)
                abuf[sl] = abuf[sl] + rbuf[sl]

        pltpu.sync_copy(abuf, deg_hbm.at[pl.ds(cid * _NP + rbase, _ROWS_SUB)])

    return _sc_gather_x_deg


@functools.cache
def _get_agg():
    @functools.partial(
        pl.kernel,
        out_type=jax.ShapeDtypeStruct((_NC * _NP, _C), jnp.float32),
        mesh=_get_mesh(),
        scratch_types=[
            pltpu.VMEM((_ECH,), jnp.int32),
            pltpu.VMEM((_ECH,), jnp.int32),
            pltpu.VMEM((_ECH,), jnp.int32),
            pltpu.VMEM((_ECH,), jnp.int32),
            pltpu.VMEM((_ECH, _C), jnp.float32),
            pltpu.VMEM((_ECH, _C), jnp.float32),
            pltpu.VMEM((_ZR, _C), jnp.float32),
            pltpu.VMEM_SHARED((_NP, _C), jnp.float32),
            pltpu.SemaphoreType.DMA,
            pltpu.SemaphoreType.DMA,
            pltpu.SemaphoreType.DMA,
            pltpu.SemaphoreType.DMA,
            pltpu.SemaphoreType.DMA,
            pltpu.SemaphoreType.DMA,
        ],
    )
    def _sc_agg(src_hbm, dst_hbm, vals_hbm, acc_hbm,
                s0, s1, d0, d1, r0, r1, zbuf, acc,
                ss0, ss1, ds0, ds1, gs0, gs1):
        cid = lax.axis_index("c")
        sid = lax.axis_index("s")
        wid = sid * _NC + cid

        z = jnp.zeros((16,), jnp.float32)

        @pl.loop(0, _ZR)
        def _(r):
            for j in range(_C // 16):
                zbuf[r, pl.ds(j * 16, 16)] = z

        rbase = sid * _ROWS_SUB
        for j in range(_ROWS_SUB // _ZR):
            pltpu.sync_copy(zbuf, acc.at[pl.ds(rbase + j * _ZR, _ZR)])
        plsc.subcore_barrier()

        ebase = wid * _EPW
        elast = ebase + _EPW - _ECH

        def load_idx(off, sb, db, ss, ds):
            o = lax.min(off, elast)
            pltpu.async_copy(src_hbm.at[pl.ds(o, _ECH)], sb, ss)
            pltpu.async_copy(dst_hbm.at[pl.ds(o, _ECH)], db, ds)

        def wait(hbm, buf, sem):
            pltpu.make_async_copy(hbm.at[pl.ds(ebase, _ECH)], buf, sem).wait()

        def gather(sb, rb, gs):
            pltpu.async_copy(vals_hbm.at[sb], rb, gs)

        load_idx(ebase, s0, d0, ss0, ds0)
        load_idx(ebase + _ECH, s1, d1, ss1, ds1)
        wait(src_hbm, s0, ss0)
        gather(s0, r0, gs0)

        def half(off, sa, da, ra, ssa, dsa, gsa, sb, db, rb, ssb, dsb, gsb):
            wait(src_hbm, sb, ssb)
            pltpu.make_async_copy(vals_hbm.at[sa], ra, gsa).wait()
            gather(sb, rb, gsb)
            wait(dst_hbm, da, dsa)
            pltpu.sync_copy(ra, acc.at[da], add=True)
            load_idx(off + 2 * _ECH, sa, da, ssa, dsa)

        @pl.loop(0, _NCH // 2)
        def _(i):
            off = ebase + 2 * i * _ECH
            half(off, s0, d0, r0, ss0, ds0, gs0, s1, d1, r1, ss1, ds1, gs1)
            half(off + _ECH, s1, d1, r1, ss1, ds1, gs1,
                 s0, d0, r0, ss0, ds0, gs0)

        wait(src_hbm, s1, ss1)
        wait(dst_hbm, d1, ds1)
        wait(dst_hbm, d0, ds0)
        pltpu.make_async_copy(vals_hbm.at[s0], r0, gs0).wait()

        plsc.subcore_barrier()
        obase = cid * _NP + rbase
        for j in range(_ROWS_SUB // _ECH):
            pltpu.sync_copy(acc.at[pl.ds(rbase + j * _ECH, _ECH)],
                            acc_hbm.at[pl.ds(obase + j * _ECH, _ECH)])

    return _sc_agg


def _tc_pre_body(x_ref, w_ref, b_ref, o_ref):
    o_ref[...] = (jnp.dot(x_ref[...], w_ref[...],
                          preferred_element_type=jnp.float32) + b_ref[...])


def _tc1_body(xr_ref, p0_ref, p1_ref, d0_ref, d1_ref, wn_ref, h_ref):
    deg = jnp.maximum(d0_ref[...] + d1_ref[...], 1.0)
    agg = (p0_ref[...] + p1_ref[...]) / deg
    h = xr_ref[...] + jnp.dot(agg, wn_ref[...],
                              preferred_element_type=jnp.float32)
    h_ref[...] = jnp.maximum(h, 0.0)


def _tc2_body(hr_ref, p0_ref, p1_ref, d0_ref, d1_ref, wn_ref,
              wh_ref, bh_ref, out_ref):
    deg = jnp.maximum(d0_ref[...] + d1_ref[...], 1.0)
    agg = (p0_ref[...] + p1_ref[...]) / deg
    h2 = hr_ref[...] + jnp.dot(agg, wn_ref[...],
                               preferred_element_type=jnp.float32)
    out_ref[...] = (jnp.dot(h2, wh_ref[...],
                            preferred_element_type=jnp.float32) + bh_ref[...])


def _row_specs():
    return [
        pl.BlockSpec((_BLK, _C), lambda i: (i, 0)),
        pl.BlockSpec((_BLK, _C), lambda i: (i, 0)),
        pl.BlockSpec((_BLK, _C), lambda i: (i + _GRID, 0)),
        pl.BlockSpec((_BLK, 1), lambda i: (i, 0)),
        pl.BlockSpec((_BLK, 1), lambda i: (i + _GRID, 0)),
        pl.BlockSpec((_C, _C), lambda i: (0, 0)),
    ]


_tc_pre = pl.pallas_call(
    _tc_pre_body,
    out_shape=jax.ShapeDtypeStruct((_NP, _C), jnp.float32),
    grid=(_GRID,),
    in_specs=[
        pl.BlockSpec((_BLK, _C), lambda i: (i, 0)),
        pl.BlockSpec((_C, _C), lambda i: (0, 0)),
        pl.BlockSpec((1, _C), lambda i: (0, 0)),
    ],
    out_specs=pl.BlockSpec((_BLK, _C), lambda i: (i, 0)),
)

_tc1 = pl.pallas_call(
    _tc1_body,
    out_shape=jax.ShapeDtypeStruct((_NP, _C), jnp.float32),
    grid=(_GRID,),
    in_specs=_row_specs(),
    out_specs=pl.BlockSpec((_BLK, _C), lambda i: (i, 0)),
)

_tc2 = pl.pallas_call(
    _tc2_body,
    out_shape=jax.ShapeDtypeStruct((_NP, _OUT), jnp.float32),
    grid=(_GRID,),
    in_specs=_row_specs() + [
        pl.BlockSpec((_C, _OUT), lambda i: (0, 0)),
        pl.BlockSpec((1, _OUT), lambda i: (0, 0)),
    ],
    out_specs=pl.BlockSpec((_BLK, _OUT), lambda i: (i, 0)),
)


def kernel(feat_table, node_idx, edge_index,
           W_root1, W_nbr1, b1, W_root2, W_nbr2, b2, W_head, b_head):
    pe = jnp.arange(_EP - _E, dtype=jnp.int32)
    nidx = jnp.concatenate(
        [node_idx, jnp.arange(_NP - _N, dtype=jnp.int32) % _T])
    src = jnp.concatenate([edge_index[0], pe % _N])
    dst = jnp.concatenate([edge_index[1], _N + pe % (_NP - _N)])

    x, deg = _get_gather_x_deg()(feat_table, nidx, dst)
    deg = deg.reshape(_NC * _NP, 1)
    xr = _tc_pre(x, W_root1, b1.reshape(1, _C))
    acc1 = _get_agg()(src, dst, x)
    h = _tc1(xr, acc1, acc1, deg, deg, W_nbr1)
    hr = _tc_pre(h, W_root2, b2.reshape(1, _C))
    acc2 = _get_agg()(src, dst, h)
    out = _tc2(hr, acc2, acc2, deg, deg, W_nbr2,
               W_head, b_head.reshape(1, _OUT))
    return out[:_N]


# speedup vs baseline: 1.0438x; 1.0438x over previous
import dataclasses
import functools

import jax
import jax.numpy as jnp
from jax import lax
from jax.experimental import pallas as pl
from jax.experimental.pallas import tpu as pltpu
from jax.experimental.pallas import tpu_sc as plsc

_N = 10000
_T = 20000
_E = 320000
_C = 128
_OUT = 10

_NC = 2
_NS = 16
_NW = _NC * _NS

_NP = 10112
_ROWS_SUB = _NP // _NS
_XCH = 128
_XNCH = _NP // _XCH
_ECH = 128
_NCH = 81
_EPW = _NCH * _ECH
_EP = _EPW * _NW

_BLK = 1264
_GRID = _NP // _BLK


@functools.cache
def _get_mesh():
    return plsc.VectorSubcoreMesh(core_axis_name="c", subcore_axis_name="s",
                                  num_cores=_NC, num_subcores=_NS)


@functools.cache
def _get_gather_x_deg():
    @functools.partial(
        pl.kernel,
        out_type=[
            jax.ShapeDtypeStruct((_NP, _C), jnp.float32),
            jax.ShapeDtypeStruct((_NC * _NP,), jnp.float32),
        ],
        mesh=_get_mesh(),
        scratch_types=[
            pltpu.VMEM((_XCH,), jnp.int32),
            pltpu.VMEM((_XCH, _C), jnp.float32),
            pltpu.VMEM((_ECH,), jnp.int32),
            pltpu.VMEM((_ECH,), jnp.int32),
            pltpu.VMEM((_NP,), jnp.float32),
            pltpu.VMEM((_ROWS_SUB,), jnp.float32),
            pltpu.VMEM((_ROWS_SUB,), jnp.float32),
            pltpu.VMEM_SHARED((_NS * _NP,), jnp.float32),
            pltpu.SemaphoreType.DMA,
            pltpu.SemaphoreType.DMA,
            pltpu.SemaphoreType.DMA,
        ],
        compiler_params=dataclasses.replace(pltpu.CompilerParams(),
                                            needs_layout_passes=False),
    )
    def _sc_gather_x_deg(tbl_hbm, nidx_hbm, dst_hbm, x_hbm, deg_hbm,
                         idx_v, rows_v, d0, d1, hbuf, rbuf, abuf, hstage,
                         sem, isem0, isem1):
        cid = lax.axis_index("c")
        sid = lax.axis_index("s")
        wid = sid * _NC + cid

        z = jnp.zeros((16,), jnp.float32)
        o = jnp.ones((16,), jnp.float32)

        @pl.loop(0, _NP // 16)
        def _(i):
            hbuf[pl.ds(i * 16, 16)] = z

        for j in range(3):
            c = wid + j * _NW

            @pl.when(c < _XNCH)
            def _():
                off = c * _XCH
                pltpu.sync_copy(nidx_hbm.at[pl.ds(off, _XCH)], idx_v)
                pltpu.async_copy(tbl_hbm.at[idx_v], rows_v, sem).wait()
                pltpu.sync_copy(rows_v, x_hbm.at[pl.ds(off, _XCH)])

        ebase = wid * _EPW
        elast = ebase + _EPW - _ECH

        def load_dst(off, buf, isem):
            pltpu.async_copy(
                dst_hbm.at[pl.ds(lax.min(off, elast), _ECH)], buf, isem)

        load_dst(ebase, d0, isem0)
        load_dst(ebase + _ECH, d1, isem1)

        def hist_half(off, da, isa):
            pltpu.make_async_copy(dst_hbm.at[pl.ds(ebase, _ECH)], da,
                                  isa).wait()
            for j in range(_ECH // 16):
                plsc.addupdate_scatter(hbuf, [da[pl.ds(j * 16, 16)]], o)
            load_dst(off + 2 * _ECH, da, isa)

        @pl.loop(0, _NCH // 3)
        def _(i):
            off = ebase + 3 * i * _ECH
            hist_half(off, d0, isem0)
            hist_half(off + _ECH, d1, isem1)
            hist_half(off + 2 * _ECH, d0, isem0)

        pltpu.make_async_copy(dst_hbm.at[pl.ds(ebase, _ECH)], d0, isem0).wait()
        pltpu.make_async_copy(dst_hbm.at[pl.ds(ebase, _ECH)], d1, isem1).wait()

        pltpu.sync_copy(hbuf, hstage.at[pl.ds(sid * _NP, _NP)])
        plsc.subcore_barrier()

        rbase = sid * _ROWS_SUB

        @pl.loop(0, _ROWS_SUB // 16)
        def _(i):
            abuf[pl.ds(i * 16, 16)] = z

        @pl.loop(0, _NS)
        def _(k):
            pltpu.sync_copy(hstage.at[pl.ds(k * _NP + rbase, _ROWS_SUB)],
                            rbuf)
            for t in range(_ROWS_SUB // 16):
                sl = pl.ds(t * 16, 16)
                abuf[sl] = abuf[sl] + rbuf[sl]

        pltpu.sync_copy(abuf, deg_hbm.at[pl.ds(cid * _NP + rbase, _ROWS_SUB)])

    return _sc_gather_x_deg


@functools.cache
def _get_agg():
    @functools.partial(
        pl.kernel,
        out_type=jax.ShapeDtypeStruct((_NC * _NP, _C), jnp.float32),
        mesh=_get_mesh(),
        scratch_types=[
            [pltpu.VMEM((_ECH,), jnp.int32) for _ in range(3)],
            [pltpu.VMEM((_ECH,), jnp.int32) for _ in range(3)],
            [pltpu.VMEM((_ECH, _C), jnp.float32) for _ in range(3)],
            pltpu.VMEM_SHARED((_NP, _C), jnp.float32),
            [pltpu.SemaphoreType.DMA for _ in range(3)],
            [pltpu.SemaphoreType.DMA for _ in range(3)],
            [pltpu.SemaphoreType.DMA for _ in range(3)],
            pltpu.SemaphoreType.DMA,
        ],
    )
    def _sc_agg(src_hbm, dst_hbm, vals_hbm, zeros_hbm, acc_hbm,
                sv, dv, rv, acc, ssem, dsem, gsem, zsem):
        cid = lax.axis_index("c")
        sid = lax.axis_index("s")
        wid = sid * _NC + cid

        rbase = sid * _ROWS_SUB
        pltpu.async_copy(zeros_hbm.at[pl.ds(rbase, _ROWS_SUB)],
                         acc.at[pl.ds(rbase, _ROWS_SUB)], zsem)

        ebase = wid * _EPW
        elast = ebase + _EPW - _ECH

        def load_idx(off, k):
            o = lax.min(off, elast)
            pltpu.async_copy(src_hbm.at[pl.ds(o, _ECH)], sv[k], ssem[k])
            pltpu.async_copy(dst_hbm.at[pl.ds(o, _ECH)], dv[k], dsem[k])

        def wait(hbm, buf, sem):
            pltpu.make_async_copy(hbm.at[pl.ds(ebase, _ECH)], buf, sem).wait()

        def gather(k):
            pltpu.async_copy(vals_hbm.at[sv[k]], rv[k], gsem[k])

        for k in range(3):
            load_idx(ebase + k * _ECH, k)
        for k in range(2):
            wait(src_hbm, sv[k], ssem[k])
            gather(k)

        pltpu.make_async_copy(zeros_hbm.at[pl.ds(rbase, _ROWS_SUB)],
                              acc.at[pl.ds(rbase, _ROWS_SUB)], zsem).wait()
        plsc.subcore_barrier()

        def body(off, a, b, c):
            wait(src_hbm, sv[c], ssem[c])
            gather(c)
            pltpu.make_async_copy(vals_hbm.at[sv[a]], rv[a], gsem[a]).wait()
            wait(dst_hbm, dv[a], dsem[a])
            pltpu.sync_copy(rv[a], acc.at[dv[a]], add=True)
            load_idx(off + 3 * _ECH, a)

        @pl.loop(0, _NCH // 3)
        def _(i):
            off = ebase + 3 * i * _ECH
            body(off, 0, 1, 2)
            body(off + _ECH, 1, 2, 0)
            body(off + 2 * _ECH, 2, 0, 1)

        wait(src_hbm, sv[2], ssem[2])
        for k in range(3):
            wait(dst_hbm, dv[k], dsem[k])
        for k in range(2):
            pltpu.make_async_copy(vals_hbm.at[sv[k]], rv[k], gsem[k]).wait()

        plsc.subcore_barrier()
        obase = cid * _NP + rbase
        pltpu.sync_copy(acc.at[pl.ds(rbase, _ROWS_SUB)],
                        acc_hbm.at[pl.ds(obase, _ROWS_SUB)])

    return _sc_agg


def _tc_pre_body(x_ref, w_ref, b_ref, o_ref):
    o_ref[...] = (jnp.dot(x_ref[...], w_ref[...],
                          preferred_element_type=jnp.float32) + b_ref[...])


def _tc1_body(xr_ref, p0_ref, p1_ref, d0_ref, d1_ref, wn_ref, h_ref):
    deg = jnp.maximum(d0_ref[...] + d1_ref[...], 1.0)
    agg = (p0_ref[...] + p1_ref[...]) / deg
    h = xr_ref[...] + jnp.dot(agg, wn_ref[...],
                              preferred_element_type=jnp.float32)
    h_ref[...] = jnp.maximum(h, 0.0)


def _tc2_body(hr_ref, p0_ref, p1_ref, d0_ref, d1_ref, wn_ref,
              wh_ref, bh_ref, out_ref):
    deg = jnp.maximum(d0_ref[...] + d1_ref[...], 1.0)
    agg = (p0_ref[...] + p1_ref[...]) / deg
    h2 = hr_ref[...] + jnp.dot(agg, wn_ref[...],
                               preferred_element_type=jnp.float32)
    out_ref[...] = (jnp.dot(h2, wh_ref[...],
                            preferred_element_type=jnp.float32) + bh_ref[...])


def _row_specs():
    return [
        pl.BlockSpec((_BLK, _C), lambda i: (i, 0)),
        pl.BlockSpec((_BLK, _C), lambda i: (i, 0)),
        pl.BlockSpec((_BLK, _C), lambda i: (i + _GRID, 0)),
        pl.BlockSpec((_BLK, 1), lambda i: (i, 0)),
        pl.BlockSpec((_BLK, 1), lambda i: (i + _GRID, 0)),
        pl.BlockSpec((_C, _C), lambda i: (0, 0)),
    ]


_tc_pre = pl.pallas_call(
    _tc_pre_body,
    out_shape=jax.ShapeDtypeStruct((_NP, _C), jnp.float32),
    grid=(_GRID,),
    in_specs=[
        pl.BlockSpec((_BLK, _C), lambda i: (i, 0)),
        pl.BlockSpec((_C, _C), lambda i: (0, 0)),
        pl.BlockSpec((1, _C), lambda i: (0, 0)),
    ],
    out_specs=pl.BlockSpec((_BLK, _C), lambda i: (i, 0)),
)

_tc1 = pl.pallas_call(
    _tc1_body,
    out_shape=jax.ShapeDtypeStruct((_NP, _C), jnp.float32),
    grid=(_GRID,),
    in_specs=_row_specs(),
    out_specs=pl.BlockSpec((_BLK, _C), lambda i: (i, 0)),
)

_tc2 = pl.pallas_call(
    _tc2_body,
    out_shape=jax.ShapeDtypeStruct((_NP, _OUT), jnp.float32),
    grid=(_GRID,),
    in_specs=_row_specs() + [
        pl.BlockSpec((_C, _OUT), lambda i: (0, 0)),
        pl.BlockSpec((1, _OUT), lambda i: (0, 0)),
    ],
    out_specs=pl.BlockSpec((_BLK, _OUT), lambda i: (i, 0)),
)


def kernel(feat_table, node_idx, edge_index,
           W_root1, W_nbr1, b1, W_root2, W_nbr2, b2, W_head, b_head):
    pe = jnp.arange(_EP - _E, dtype=jnp.int32)
    nidx = jnp.concatenate(
        [node_idx, jnp.arange(_NP - _N, dtype=jnp.int32) % _T])
    src = jnp.concatenate([edge_index[0], pe % _N])
    dst = jnp.concatenate([edge_index[1], _N + pe % (_NP - _N)])
    zeros = jnp.zeros((_NP, _C), jnp.float32)

    x, deg = _get_gather_x_deg()(feat_table, nidx, dst)
    deg = deg.reshape(_NC * _NP, 1)
    xr = _tc_pre(x, W_root1, b1.reshape(1, _C))
    acc1 = _get_agg()(src, dst, x, zeros)
    h = _tc1(xr, acc1, acc1, deg, deg, W_nbr1)
    hr = _tc_pre(h, W_root2, b2.reshape(1, _C))
    acc2 = _get_agg()(src, dst, h, zeros)
    out = _tc2(hr, acc2, acc2, deg, deg, W_nbr2,
               W_head, b_head.reshape(1, _OUT))
    return out[:_N]
